# baseline (speedup 1.0000x reference)
import jax
import jax.numpy as jnp
from jax import lax
from jax.experimental import pallas as pl
from jax.experimental.pallas import tpu as pltpu
from jax.experimental.pallas import tpu_sc as plsc

N_NODES = 10000
N_EDGES = 320000
D = 128

NC = 2
NS = 16
HALF = 5120
N_PAD = NC * HALF
ACC_ROWS = HALF + 64
CHUNK = 128
NCHUNK = 160
EDGES_PAD = NS * NCHUNK * CHUNK
DST_PAD = 10016
NBUF = 4
BLK = 8
NBLK = NCHUNK // BLK
ZCH = 64
WB_ROWS = 80


def _spmm_body(src_hbm, dst_hbm, x_hbm, out_hbm,
               idx_s, dblk0, dblk1, rows0, rows1, rows2, rows3,
               acc_sp, g0, g1, g2, g3, d0, d1):
    c = lax.axis_index("c")
    s = lax.axis_index("s")
    wid = c * NS + s
    bufs = [rows0, rows1, rows2, rows3]
    gsems = [g0, g1, g2, g3]
    dblks = [dblk0, dblk1]
    dsems = [d0, d1]

    zero16 = jnp.zeros((16,), jnp.float32)
    for r in range(ZCH):
        for l in range(D // 16):
            rows0[r, pl.ds(l * 16, 16)] = zero16
    zsrc = rows0.at[pl.ds(0, ZCH)]

    def zero_step(k, _):
        pltpu.sync_copy(zsrc, acc_sp.at[pl.ds((s + k * NS) * ZCH, ZCH)])
        return 0
    lax.fori_loop(0, HALF // ZCH // NS, zero_step, 0)

    pltpu.sync_copy(src_hbm.at[s], idx_s)
    pltpu.async_copy(dst_hbm.at[wid, 0], dblk0, d0)

    plsc.subcore_barrier()

    for k in range(NBUF):
        pltpu.async_copy(x_hbm.at[idx_s.at[k]], bufs[k], gsems[k])

    def super_step(sb, _):
        for p in range(2):
            b = 2 * sb + p

            @pl.when(b + 1 < NBLK)
            def _():
                pltpu.async_copy(dst_hbm.at[wid, b + 1], dblks[1 - p],
                                 dsems[1 - p])
            pltpu.make_async_copy(dst_hbm.at[wid, b], dblks[p],
                                  dsems[p]).wait()
            for q in range(BLK):
                j = b * BLK + q
                k = q % NBUF
                pltpu.make_async_copy(x_hbm.at[idx_s.at[j]], bufs[k],
                                      gsems[k]).wait()
                pltpu.sync_copy(bufs[k], acc_sp.at[dblks[p].at[q]],
                                add=True)

                @pl.when(j + NBUF < NCHUNK)
                def _():
                    pltpu.async_copy(x_hbm.at[idx_s.at[j + NBUF]],
                                     bufs[k], gsems[k])
        return 0
    lax.fori_loop(0, NBLK // 2, super_step, 0)

    plsc.subcore_barrier()

    wsrc = rows0.at[pl.ds(0, WB_ROWS)]

    def wb_step(k, _):
        base = s * (HALF // NS) + k * WB_ROWS
        pltpu.sync_copy(acc_sp.at[pl.ds(base, WB_ROWS)], wsrc)
        pltpu.sync_copy(wsrc, out_hbm.at[pl.ds(c * HALF + base, WB_ROWS)])
        return 0
    lax.fori_loop(0, HALF // NS // WB_ROWS, wb_step, 0)


def _sc_spmm(x, src3, dst4):
    mesh = plsc.VectorSubcoreMesh(core_axis_name="c", subcore_axis_name="s")
    return pl.kernel(
        _spmm_body,
        out_type=jax.ShapeDtypeStruct((N_PAD, D), jnp.float32),
        mesh=mesh,
        scratch_types=[
            pltpu.VMEM((NCHUNK, CHUNK), jnp.int32),
            pltpu.VMEM((BLK, CHUNK), jnp.int32),
            pltpu.VMEM((BLK, CHUNK), jnp.int32),
            pltpu.VMEM((CHUNK, D), jnp.float32),
            pltpu.VMEM((CHUNK, D), jnp.float32),
            pltpu.VMEM((CHUNK, D), jnp.float32),
            pltpu.VMEM((CHUNK, D), jnp.float32),
            pltpu.VMEM_SHARED((ACC_ROWS, D), jnp.float32),
            pltpu.SemaphoreType.DMA,
            pltpu.SemaphoreType.DMA,
            pltpu.SemaphoreType.DMA,
            pltpu.SemaphoreType.DMA,
            pltpu.SemaphoreType.DMA,
            pltpu.SemaphoreType.DMA,
        ],
    )(src3, dst4, x)


ROW_BLK = 1000


def _layer_body(p_ref, w_ref, b_ref, out_ref):
    y = lax.dot_general(p_ref[...], w_ref[...], (((1,), (1,)), ((), ())),
                        preferred_element_type=jnp.float32)
    out_ref[...] = jnp.maximum(y + b_ref[...], 0.0)


def _tc_layer(p, w, b):
    grid = N_NODES // ROW_BLK
    return pl.pallas_call(
        _layer_body,
        grid=(grid,),
        in_specs=[
            pl.BlockSpec((ROW_BLK, D), lambda i: (i, 0)),
            pl.BlockSpec((D, D), lambda i: (0, 0)),
            pl.BlockSpec((1, D), lambda i: (0, 0)),
        ],
        out_specs=pl.BlockSpec((ROW_BLK, D), lambda i: (i, 0)),
        out_shape=jax.ShapeDtypeStruct((N_NODES, D), jnp.float32),
    )(p[:N_NODES], w, b.reshape(1, D))


def _head_body(h_ref, wd1_ref, bd1_ref, wd2_ref, bd2_ref,
               out_ref, acc_ref):
    i = pl.program_id(0)

    @pl.when(i == 0)
    def _():
        acc_ref[...] = jnp.zeros_like(acc_ref)

    acc_ref[...] += jnp.sum(h_ref[...], axis=0, keepdims=True)

    @pl.when(i == pl.num_programs(0) - 1)
    def _():
        emb = acc_ref[...] * (1.0 / N_NODES)
        d = lax.dot_general(emb, wd1_ref[...], (((1,), (1,)), ((), ())),
                            preferred_element_type=jnp.float32)
        d = jnp.maximum(d + bd1_ref[...], 0.0)
        z = lax.dot_general(d, wd2_ref[...], (((1,), (1,)), ((), ())),
                            preferred_element_type=jnp.float32)
        out_ref[...] = jax.nn.sigmoid(z + bd2_ref[...])


def _tc_head(h, wd1, bd1, wd2, bd2):
    grid = N_NODES // ROW_BLK
    full = lambda i: (0, 0)
    return pl.pallas_call(
        _head_body,
        grid=(grid,),
        in_specs=[
            pl.BlockSpec((ROW_BLK, D), lambda i: (i, 0)),
            pl.BlockSpec((D, D), full),
            pl.BlockSpec((1, D), full),
            pl.BlockSpec((D, D), full),
            pl.BlockSpec((1, D), full),
        ],
        out_specs=pl.BlockSpec((1, D), full),
        out_shape=jax.ShapeDtypeStruct((1, D), jnp.float32),
        scratch_shapes=[pltpu.VMEM((1, D), jnp.float32)],
    )(h, wd1, bd1.reshape(1, D), wd2, bd2.reshape(1, D))


def kernel(node_features, edge_index, W1, b1, W2, b2, Wd1, bd1, Wd2, bd2):
    npad = EDGES_PAD - N_EDGES
    src = jnp.concatenate(
        [edge_index[0].astype(jnp.int32),
         jnp.zeros((npad,), jnp.int32)]).reshape(NS, NCHUNK, CHUNK)
    d = jnp.concatenate(
        [edge_index[1].astype(jnp.int32),
         jnp.full((npad,), DST_PAD, jnp.int32)])
    trash = HALF + (d & 63)
    l0 = jnp.where(d >= HALF, trash, d)
    l1w = d - HALF
    l1 = jnp.where(l1w < 0, trash, l1w)
    dst = jnp.stack([l0, l1]).reshape(NC * NS, NBLK, BLK, CHUNK)

    p1 = _sc_spmm(node_features, src, dst)
    h1 = _tc_layer(p1, W1, b1)
    p2 = _sc_spmm(h1, src, dst)
    h2 = _tc_layer(p2, W2, b2)
    policy = _tc_head(h2, Wd1, bd1, Wd2, bd2)
    return policy.reshape(D)


# speedup vs baseline: 2.1093x; 2.1093x over previous
import jax
import jax.numpy as jnp
from jax import lax
from jax.experimental import pallas as pl
from jax.experimental.pallas import tpu as pltpu
from jax.experimental.pallas import tpu_sc as plsc

N_NODES = 10000
N_EDGES = 320000
D = 128

NC = 2
NS = 16
XW = D // NC
N_PAD = 10240
ACC_ROWS = N_PAD
CHUNK = 128
NCHUNK = 160
EDGES_PAD = NS * NCHUNK * CHUNK
DST_PAD = 10016
NBUF = 4
BLK = 8
NBLK = NCHUNK // BLK
ZCH = 64
WB_ROWS = 128


def _spmm_body(src_hbm, dst_hbm, x2_hbm, out_hbm,
               idx_s, dblk0, dblk1, rows0, rows1, rows2, rows3,
               acc_sp, g0, g1, g2, g3, d0, d1):
    c = lax.axis_index("c")
    s = lax.axis_index("s")
    bufs = [rows0, rows1, rows2, rows3]
    gsems = [g0, g1, g2, g3]
    dblks = [dblk0, dblk1]
    dsems = [d0, d1]
    xc = x2_hbm.at[c]

    zero16 = jnp.zeros((16,), jnp.float32)
    for r in range(ZCH):
        for l in range(XW // 16):
            rows0[r, pl.ds(l * 16, 16)] = zero16
    zsrc = rows0.at[pl.ds(0, ZCH)]

    def zero_step(k, _):
        pltpu.sync_copy(zsrc,
                        acc_sp.at[pl.ds((s * (N_PAD // NS // ZCH) + k) * ZCH,
                                        ZCH)])
        return 0
    lax.fori_loop(0, N_PAD // NS // ZCH, zero_step, 0)

    pltpu.sync_copy(src_hbm.at[s], idx_s)
    pltpu.async_copy(dst_hbm.at[s, 0], dblk0, d0)

    plsc.subcore_barrier()

    for k in range(NBUF):
        pltpu.async_copy(xc.at[idx_s.at[k]], bufs[k], gsems[k])

    def super_step(sb, _):
        for p in range(2):
            b = 2 * sb + p

            @pl.when(b + 1 < NBLK)
            def _():
                pltpu.async_copy(dst_hbm.at[s, b + 1], dblks[1 - p],
                                 dsems[1 - p])
            pltpu.make_async_copy(dst_hbm.at[s, b], dblks[p],
                                  dsems[p]).wait()
            for q in range(BLK):
                j = b * BLK + q
                k = q % NBUF
                pltpu.make_async_copy(xc.at[idx_s.at[j]], bufs[k],
                                      gsems[k]).wait()
                pltpu.sync_copy(bufs[k], acc_sp.at[dblks[p].at[q]],
                                add=True)

                @pl.when(j + NBUF < NCHUNK)
                def _():
                    pltpu.async_copy(xc.at[idx_s.at[j + NBUF]],
                                     bufs[k], gsems[k])
        return 0
    lax.fori_loop(0, NBLK // 2, super_step, 0)

    plsc.subcore_barrier()

    wsrc = rows0.at[pl.ds(0, WB_ROWS)]

    def wb_step(k, _):
        base = s * (N_PAD // NS) + k * WB_ROWS
        pltpu.sync_copy(acc_sp.at[pl.ds(base, WB_ROWS)], wsrc)
        pltpu.sync_copy(wsrc, out_hbm.at[c, pl.ds(base, WB_ROWS)])
        return 0
    lax.fori_loop(0, N_PAD // NS // WB_ROWS, wb_step, 0)


def _sc_spmm(x2, src3, dst4):
    mesh = plsc.VectorSubcoreMesh(core_axis_name="c", subcore_axis_name="s")
    return pl.kernel(
        _spmm_body,
        out_type=jax.ShapeDtypeStruct((NC, N_PAD, XW), jnp.float32),
        mesh=mesh,
        compiler_params=pltpu.CompilerParams(use_tc_tiling_on_sc=False),
        scratch_types=[
            pltpu.VMEM((NCHUNK, CHUNK), jnp.int32),
            pltpu.VMEM((BLK, CHUNK), jnp.int32),
            pltpu.VMEM((BLK, CHUNK), jnp.int32),
            pltpu.VMEM((CHUNK, XW), jnp.float32),
            pltpu.VMEM((CHUNK, XW), jnp.float32),
            pltpu.VMEM((CHUNK, XW), jnp.float32),
            pltpu.VMEM((CHUNK, XW), jnp.float32),
            pltpu.VMEM_SHARED((ACC_ROWS, XW), jnp.float32),
            pltpu.SemaphoreType.DMA,
            pltpu.SemaphoreType.DMA,
            pltpu.SemaphoreType.DMA,
            pltpu.SemaphoreType.DMA,
            pltpu.SemaphoreType.DMA,
            pltpu.SemaphoreType.DMA,
        ],
    )(src3, dst4, x2)


ROW_BLK = 1000


def _layer_body(p_ref, w_ref, b_ref, out_ref):
    y = lax.dot_general(p_ref[...], w_ref[...], (((1,), (1,)), ((), ())),
                        preferred_element_type=jnp.float32)
    out_ref[...] = jnp.maximum(y + b_ref[...], 0.0)


def _tc_layer(p, w, b):
    grid = N_NODES // ROW_BLK
    return pl.pallas_call(
        _layer_body,
        grid=(grid,),
        in_specs=[
            pl.BlockSpec((ROW_BLK, D), lambda i: (i, 0)),
            pl.BlockSpec((D, D), lambda i: (0, 0)),
            pl.BlockSpec((1, D), lambda i: (0, 0)),
        ],
        out_specs=pl.BlockSpec((ROW_BLK, D), lambda i: (i, 0)),
        out_shape=jax.ShapeDtypeStruct((N_NODES, D), jnp.float32),
    )(p[:N_NODES], w, b.reshape(1, D))


def _head_body(h_ref, wd1_ref, bd1_ref, wd2_ref, bd2_ref,
               out_ref, acc_ref):
    i = pl.program_id(0)

    @pl.when(i == 0)
    def _():
        acc_ref[...] = jnp.zeros_like(acc_ref)

    acc_ref[...] += jnp.sum(h_ref[...], axis=0, keepdims=True)

    @pl.when(i == pl.num_programs(0) - 1)
    def _():
        emb = acc_ref[...] * (1.0 / N_NODES)
        d = lax.dot_general(emb, wd1_ref[...], (((1,), (1,)), ((), ())),
                            preferred_element_type=jnp.float32)
        d = jnp.maximum(d + bd1_ref[...], 0.0)
        z = lax.dot_general(d, wd2_ref[...], (((1,), (1,)), ((), ())),
                            preferred_element_type=jnp.float32)
        out_ref[...] = jax.nn.sigmoid(z + bd2_ref[...])


def _tc_head(h, wd1, bd1, wd2, bd2):
    grid = N_NODES // ROW_BLK
    full = lambda i: (0, 0)
    return pl.pallas_call(
        _head_body,
        grid=(grid,),
        in_specs=[
            pl.BlockSpec((ROW_BLK, D), lambda i: (i, 0)),
            pl.BlockSpec((D, D), full),
            pl.BlockSpec((1, D), full),
            pl.BlockSpec((D, D), full),
            pl.BlockSpec((1, D), full),
        ],
        out_specs=pl.BlockSpec((1, D), full),
        out_shape=jax.ShapeDtypeStruct((1, D), jnp.float32),
        scratch_shapes=[pltpu.VMEM((1, D), jnp.float32)],
    )(h, wd1, bd1.reshape(1, D), wd2, bd2.reshape(1, D))


def kernel(node_features, edge_index, W1, b1, W2, b2, Wd1, bd1, Wd2, bd2):
    npad = EDGES_PAD - N_EDGES
    src = jnp.concatenate(
        [edge_index[0].astype(jnp.int32),
         jnp.zeros((npad,), jnp.int32)]).reshape(NS, NCHUNK, CHUNK)
    dst = jnp.concatenate(
        [edge_index[1].astype(jnp.int32),
         jnp.full((npad,), DST_PAD, jnp.int32)]).reshape(NS, NBLK, BLK, CHUNK)

    def spmm(h):
        h2 = jnp.stack([h[:, :XW], h[:, XW:]])
        o = _sc_spmm(h2, src, dst)
        return jnp.concatenate([o[0, :N_NODES], o[1, :N_NODES]], axis=1)

    h1 = _tc_layer(spmm(node_features), W1, b1)
    h2 = _tc_layer(spmm(h1), W2, b2)
    policy = _tc_head(h2, Wd1, bd1, Wd2, bd2)
    return policy.reshape(D)


# speedup vs baseline: 3.8467x; 1.8237x over previous
import jax
import jax.numpy as jnp
from jax import lax
from jax.experimental import pallas as pl
from jax.experimental.pallas import tpu as pltpu
from jax.experimental.pallas import tpu_sc as plsc

N_NODES = 10000
N_EDGES = 320000
D = 128

NC = 2
NS = 16
XW = D // NC
N_PAD = 10240
ACC_ROWS = N_PAD
CHUNK = 128
NCHUNK = 160
EDGES_PAD = NS * NCHUNK * CHUNK
DST_PAD = 10016
NBUF = 2
GLEAD = 2
BLK = 8
NBLK = NCHUNK // BLK
ZCH = 64
WB_ROWS = 128


def _spmm_body(src_hbm, dst_hbm, x2_hbm, out_hbm,
               idx_s, dblk0, dblk1, rows0, rows1,
               acc_sp, x_sp, g0, g1, d0, d1):
    c = lax.axis_index("c")
    s = lax.axis_index("s")
    bufs = [rows0, rows1]
    gsems = [g0, g1]
    dblks = [dblk0, dblk1]
    dsems = [d0, d1]
    xc = x2_hbm.at[c]

    zero16 = jnp.zeros((16,), jnp.float32)
    for r in range(ZCH):
        for l in range(XW // 16):
            rows0[r, pl.ds(l * 16, 16)] = zero16
    zsrc = rows0.at[pl.ds(0, ZCH)]

    def zero_step(k, _):
        pltpu.sync_copy(zsrc,
                        acc_sp.at[pl.ds((s * (N_PAD // NS // ZCH) + k) * ZCH,
                                        ZCH)])
        return 0
    lax.fori_loop(0, N_PAD // NS // ZCH, zero_step, 0)

    def xst_step(k, _):
        base = s * (N_PAD // NS) + k * WB_ROWS

        @pl.when(base + WB_ROWS <= N_NODES)
        def _():
            pltpu.sync_copy(xc.at[pl.ds(base, WB_ROWS)], rows1)
            pltpu.sync_copy(rows1, x_sp.at[pl.ds(base, WB_ROWS)])
        return 0
    lax.fori_loop(0, N_PAD // NS // WB_ROWS, xst_step, 0)

    @pl.when(s == NS - 1)
    def _():
        xr = N_NODES % WB_ROWS
        xtail = rows1.at[pl.ds(0, xr)]
        pltpu.sync_copy(xc.at[pl.ds(N_NODES - xr, xr)], xtail)
        pltpu.sync_copy(xtail, x_sp.at[pl.ds(N_NODES - xr, xr)])

    pltpu.sync_copy(src_hbm.at[s], idx_s)
    pltpu.async_copy(dst_hbm.at[s, 0], dblk0, d0)

    plsc.subcore_barrier()

    for k in range(NBUF):
        pltpu.async_copy(x_sp.at[idx_s.at[k]], bufs[k], gsems[k])

    def super_step(sb, _):
        for p in range(2):
            b = 2 * sb + p
            pltpu.make_async_copy(dst_hbm.at[s, b], dblks[p],
                                  dsems[p]).wait()
            for q in range(BLK):
                j = b * BLK + q
                k = q % NBUF
                pltpu.make_async_copy(x_sp.at[idx_s.at[j]], bufs[k],
                                      gsems[k]).wait()
                pltpu.sync_copy(bufs[k], acc_sp.at[dblks[p].at[q]],
                                add=True)

                @pl.when(j + NBUF < NCHUNK)
                def _():
                    pltpu.async_copy(x_sp.at[idx_s.at[j + NBUF]],
                                     bufs[k], gsems[k])

                if q == 2:
                    @pl.when(b + 1 < NBLK)
                    def _():
                        pltpu.async_copy(dst_hbm.at[s, b + 1],
                                         dblks[1 - p], dsems[1 - p])
        return 0
    lax.fori_loop(0, NBLK // 2, super_step, 0)

    plsc.subcore_barrier()

    wsrc = rows0.at[pl.ds(0, WB_ROWS)]

    def wb_step(k, _):
        base = s * (N_PAD // NS) + k * WB_ROWS

        @pl.when(base + WB_ROWS <= N_NODES)
        def _():
            pltpu.sync_copy(acc_sp.at[pl.ds(base, WB_ROWS)], wsrc)
            pltpu.sync_copy(wsrc, out_hbm.at[c, pl.ds(base, WB_ROWS)])
        return 0
    lax.fori_loop(0, N_PAD // NS // WB_ROWS, wb_step, 0)

    @pl.when(s == NS - 1)
    def _():
        tail = rows0.at[pl.ds(0, N_NODES % WB_ROWS)]
        pltpu.sync_copy(acc_sp.at[pl.ds(N_NODES - N_NODES % WB_ROWS,
                                        N_NODES % WB_ROWS)], tail)
        pltpu.sync_copy(tail, out_hbm.at[c, pl.ds(N_NODES - N_NODES % WB_ROWS,
                                                  N_NODES % WB_ROWS)])


def _sc_spmm(x2, src3, dst4):
    mesh = plsc.VectorSubcoreMesh(core_axis_name="c", subcore_axis_name="s")
    rowbuf = pltpu.VMEM((CHUNK, XW), jnp.float32)
    dma = pltpu.SemaphoreType.DMA
    return pl.kernel(
        _spmm_body,
        out_type=jax.ShapeDtypeStruct((NC, N_NODES, XW), jnp.float32),
        mesh=mesh,
        compiler_params=pltpu.CompilerParams(use_tc_tiling_on_sc=False),
        scratch_types=[
            pltpu.VMEM((NCHUNK, CHUNK), jnp.int32),
            pltpu.VMEM((BLK, CHUNK), jnp.int32),
            pltpu.VMEM((BLK, CHUNK), jnp.int32),
        ] + [rowbuf] * NBUF + [
            pltpu.VMEM_SHARED((ACC_ROWS, XW), jnp.float32),
            pltpu.VMEM_SHARED((N_NODES, XW), jnp.float32),
        ] + [dma] * (NBUF + 2),
    )(src3, dst4, x2)


ROW_BLK = 1000


def _layer_body(o0_ref, o1_ref, w_ref, b_ref, out_ref):
    w = w_ref[...]
    y = lax.dot_general(o0_ref[0], w[:, :XW], (((1,), (1,)), ((), ())),
                        preferred_element_type=jnp.float32)
    y += lax.dot_general(o1_ref[0], w[:, XW:], (((1,), (1,)), ((), ())),
                         preferred_element_type=jnp.float32)
    h = jnp.maximum(y + b_ref[...], 0.0)
    out_ref[0] = h[:, :XW]
    out_ref[1] = h[:, XW:]


def _tc_layer(o, w, b):
    grid = N_NODES // ROW_BLK
    return pl.pallas_call(
        _layer_body,
        grid=(grid,),
        in_specs=[
            pl.BlockSpec((1, ROW_BLK, XW), lambda i: (0, i, 0)),
            pl.BlockSpec((1, ROW_BLK, XW), lambda i: (1, i, 0)),
            pl.BlockSpec((D, D), lambda i: (0, 0)),
            pl.BlockSpec((1, D), lambda i: (0, 0)),
        ],
        out_specs=pl.BlockSpec((NC, ROW_BLK, XW), lambda i: (0, i, 0)),
        out_shape=jax.ShapeDtypeStruct((NC, N_NODES, XW), jnp.float32),
    )(o, o, w, b.reshape(1, D))


def _head_body(o0_ref, o1_ref, w2_ref, b2_ref, wd1_ref, bd1_ref,
               wd2_ref, bd2_ref, out_ref, acc_ref):
    i = pl.program_id(0)

    @pl.when(i == 0)
    def _():
        acc_ref[...] = jnp.zeros_like(acc_ref)

    w2 = w2_ref[...]
    y = lax.dot_general(o0_ref[0], w2[:, :XW], (((1,), (1,)), ((), ())),
                        preferred_element_type=jnp.float32)
    y += lax.dot_general(o1_ref[0], w2[:, XW:], (((1,), (1,)), ((), ())),
                         preferred_element_type=jnp.float32)
    h2 = jnp.maximum(y + b2_ref[...], 0.0)
    acc_ref[...] += jnp.sum(h2, axis=0, keepdims=True)

    @pl.when(i == pl.num_programs(0) - 1)
    def _():
        emb = acc_ref[...] * (1.0 / N_NODES)
        d = lax.dot_general(emb, wd1_ref[...], (((1,), (1,)), ((), ())),
                            preferred_element_type=jnp.float32)
        d = jnp.maximum(d + bd1_ref[...], 0.0)
        z = lax.dot_general(d, wd2_ref[...], (((1,), (1,)), ((), ())),
                            preferred_element_type=jnp.float32)
        out_ref[...] = jax.nn.sigmoid(z + bd2_ref[...])


def _tc_head(o, w2, b2, wd1, bd1, wd2, bd2):
    grid = N_NODES // ROW_BLK
    full = lambda i: (0, 0)
    return pl.pallas_call(
        _head_body,
        grid=(grid,),
        in_specs=[
            pl.BlockSpec((1, ROW_BLK, XW), lambda i: (0, i, 0)),
            pl.BlockSpec((1, ROW_BLK, XW), lambda i: (1, i, 0)),
            pl.BlockSpec((D, D), full),
            pl.BlockSpec((1, D), full),
            pl.BlockSpec((D, D), full),
            pl.BlockSpec((1, D), full),
            pl.BlockSpec((D, D), full),
            pl.BlockSpec((1, D), full),
        ],
        out_specs=pl.BlockSpec((1, D), full),
        out_shape=jax.ShapeDtypeStruct((1, D), jnp.float32),
        scratch_shapes=[pltpu.VMEM((1, D), jnp.float32)],
    )(o, o, w2, b2.reshape(1, D),
      wd1, bd1.reshape(1, D), wd2, bd2.reshape(1, D))


def kernel(node_features, edge_index, W1, b1, W2, b2, Wd1, bd1, Wd2, bd2):
    npad = EDGES_PAD - N_EDGES
    src = jnp.concatenate(
        [edge_index[0].astype(jnp.int32),
         jnp.zeros((npad,), jnp.int32)]).reshape(NS, NCHUNK, CHUNK)
    dst = jnp.concatenate(
        [edge_index[1].astype(jnp.int32),
         jnp.full((npad,), DST_PAD, jnp.int32)]).reshape(NS, NBLK, BLK, CHUNK)

    x2 = jnp.stack([node_features[:, :XW], node_features[:, XW:]])
    o1 = _sc_spmm(x2, src, dst)
    h1 = _tc_layer(o1, W1, b1)
    o2 = _sc_spmm(h1, src, dst)
    policy = _tc_head(o2, W2, b2, Wd1, bd1, Wd2, bd2)
    return policy.reshape(D)


# speedup vs baseline: 4.3892x; 1.1410x over previous
import jax
import jax.numpy as jnp
from jax import lax
from jax.experimental import pallas as pl
from jax.experimental.pallas import tpu as pltpu
from jax.experimental.pallas import tpu_sc as plsc

N_NODES = 10000
N_EDGES = 320000
D = 128

NC = 2
NS = 16
XW = D // NC
N_PAD = 10240
ACC_ROWS = N_PAD
CHUNK = 128
NCHUNK = 160
EDGES_PAD = NS * NCHUNK * CHUNK
DST_PAD = 10016
NBUF = 2
GLEAD = 2
BLK = 8
NBLK = NCHUNK // BLK
ZCH = 64
WB_ROWS = 128


def _spmm_body(src_hbm, dst_hbm, x2_hbm, out_hbm,
               idx_s, dblk0, dblk1, rows0, rows1,
               acc_sp, x_sp, g0, g1, d0, d1):
    c = lax.axis_index("c")
    s = lax.axis_index("s")
    bufs = [rows0, rows1]
    gsems = [g0, g1]
    dblks = [dblk0, dblk1]
    dsems = [d0, d1]
    col0 = c * XW

    zero16 = jnp.zeros((16,), jnp.float32)
    for r in range(ZCH):
        for l in range(XW // 16):
            rows0[r, pl.ds(l * 16, 16)] = zero16
    zsrc = rows0.at[pl.ds(0, ZCH)]

    def zero_step(k, _):
        pltpu.sync_copy(zsrc,
                        acc_sp.at[pl.ds((s * (N_PAD // NS // ZCH) + k) * ZCH,
                                        ZCH)])
        return 0
    lax.fori_loop(0, N_PAD // NS // ZCH, zero_step, 0)

    def xst_step(k, _):
        base = s * (N_PAD // NS) + k * WB_ROWS

        @pl.when(base + WB_ROWS <= N_NODES)
        def _():
            pltpu.sync_copy(x2_hbm.at[pl.ds(base, WB_ROWS),
                                      pl.ds(col0, XW)], rows1)
            pltpu.sync_copy(rows1, x_sp.at[pl.ds(base, WB_ROWS)])
        return 0
    lax.fori_loop(0, N_PAD // NS // WB_ROWS, xst_step, 0)

    @pl.when(s == NS - 1)
    def _():
        xr = N_NODES % WB_ROWS
        xtail = rows1.at[pl.ds(0, xr)]
        pltpu.sync_copy(x2_hbm.at[pl.ds(N_NODES - xr, xr),
                                  pl.ds(col0, XW)], xtail)
        pltpu.sync_copy(xtail, x_sp.at[pl.ds(N_NODES - xr, xr)])

    pltpu.sync_copy(src_hbm.at[s], idx_s)
    pltpu.async_copy(dst_hbm.at[s, 0], dblk0, d0)

    plsc.subcore_barrier()

    for k in range(NBUF):
        pltpu.async_copy(x_sp.at[idx_s.at[k]], bufs[k], gsems[k])

    def super_step(sb, _):
        for p in range(2):
            b = 2 * sb + p
            pltpu.make_async_copy(dst_hbm.at[s, b], dblks[p],
                                  dsems[p]).wait()
            for q in range(BLK):
                j = b * BLK + q
                k = q % NBUF
                pltpu.make_async_copy(x_sp.at[idx_s.at[j]], bufs[k],
                                      gsems[k]).wait()
                pltpu.sync_copy(bufs[k], acc_sp.at[dblks[p].at[q]],
                                add=True)

                @pl.when(j + NBUF < NCHUNK)
                def _():
                    pltpu.async_copy(x_sp.at[idx_s.at[j + NBUF]],
                                     bufs[k], gsems[k])

                if q == 2:
                    @pl.when(b + 1 < NBLK)
                    def _():
                        pltpu.async_copy(dst_hbm.at[s, b + 1],
                                         dblks[1 - p], dsems[1 - p])
        return 0
    lax.fori_loop(0, NBLK // 2, super_step, 0)

    plsc.subcore_barrier()

    wsrc = rows0.at[pl.ds(0, WB_ROWS)]

    def wb_step(k, _):
        base = s * (N_PAD // NS) + k * WB_ROWS

        @pl.when(base + WB_ROWS <= N_NODES)
        def _():
            pltpu.sync_copy(acc_sp.at[pl.ds(base, WB_ROWS)], wsrc)
            pltpu.sync_copy(wsrc, out_hbm.at[pl.ds(base, WB_ROWS),
                                             pl.ds(col0, XW)])
        return 0
    lax.fori_loop(0, N_PAD // NS // WB_ROWS, wb_step, 0)

    @pl.when(s == NS - 1)
    def _():
        nr = N_NODES % WB_ROWS
        tail = rows0.at[pl.ds(0, nr)]
        pltpu.sync_copy(acc_sp.at[pl.ds(N_NODES - nr, nr)], tail)
        pltpu.sync_copy(tail, out_hbm.at[pl.ds(N_NODES - nr, nr),
                                         pl.ds(col0, XW)])


def _sc_spmm(x2, src3, dst4):
    mesh = plsc.VectorSubcoreMesh(core_axis_name="c", subcore_axis_name="s")
    rowbuf = pltpu.VMEM((CHUNK, XW), jnp.float32)
    dma = pltpu.SemaphoreType.DMA
    return pl.kernel(
        _spmm_body,
        out_type=jax.ShapeDtypeStruct((N_NODES, D), jnp.float32),
        mesh=mesh,
        compiler_params=pltpu.CompilerParams(use_tc_tiling_on_sc=False),
        scratch_types=[
            pltpu.VMEM((NCHUNK, CHUNK), jnp.int32),
            pltpu.VMEM((BLK, CHUNK), jnp.int32),
            pltpu.VMEM((BLK, CHUNK), jnp.int32),
        ] + [rowbuf] * NBUF + [
            pltpu.VMEM_SHARED((ACC_ROWS, XW), jnp.float32),
            pltpu.VMEM_SHARED((N_NODES, XW), jnp.float32),
        ] + [dma] * (NBUF + 2),
    )(src3, dst4, x2)


ROW_BLK = 1000


def _layer_body(p_ref, w_ref, b_ref, out_ref):
    y = lax.dot_general(p_ref[...], w_ref[...], (((1,), (1,)), ((), ())),
                        preferred_element_type=jnp.float32)
    out_ref[...] = jnp.maximum(y + b_ref[...], 0.0)


def _tc_layer(p, w, b):
    grid = N_NODES // ROW_BLK
    return pl.pallas_call(
        _layer_body,
        grid=(grid,),
        in_specs=[
            pl.BlockSpec((ROW_BLK, D), lambda i: (i, 0)),
            pl.BlockSpec((D, D), lambda i: (0, 0)),
            pl.BlockSpec((1, D), lambda i: (0, 0)),
        ],
        out_specs=pl.BlockSpec((ROW_BLK, D), lambda i: (i, 0)),
        out_shape=jax.ShapeDtypeStruct((N_NODES, D), jnp.float32),
    )(p, w, b.reshape(1, D))


def _head_body(p_ref, w2_ref, b2_ref, wd1_ref, bd1_ref,
               wd2_ref, bd2_ref, out_ref, acc_ref):
    i = pl.program_id(0)

    @pl.when(i == 0)
    def _():
        acc_ref[...] = jnp.zeros_like(acc_ref)

    y = lax.dot_general(p_ref[...], w2_ref[...], (((1,), (1,)), ((), ())),
                        preferred_element_type=jnp.float32)
    h2 = jnp.maximum(y + b2_ref[...], 0.0)
    acc_ref[...] += jnp.sum(h2, axis=0, keepdims=True)

    @pl.when(i == pl.num_programs(0) - 1)
    def _():
        emb = acc_ref[...] * (1.0 / N_NODES)
        d = lax.dot_general(emb, wd1_ref[...], (((1,), (1,)), ((), ())),
                            preferred_element_type=jnp.float32)
        d = jnp.maximum(d + bd1_ref[...], 0.0)
        z = lax.dot_general(d, wd2_ref[...], (((1,), (1,)), ((), ())),
                            preferred_element_type=jnp.float32)
        out_ref[...] = jax.nn.sigmoid(z + bd2_ref[...])


def _tc_head(p, w2, b2, wd1, bd1, wd2, bd2):
    grid = N_NODES // ROW_BLK
    full = lambda i: (0, 0)
    return pl.pallas_call(
        _head_body,
        grid=(grid,),
        in_specs=[
            pl.BlockSpec((ROW_BLK, D), lambda i: (i, 0)),
            pl.BlockSpec((D, D), full),
            pl.BlockSpec((1, D), full),
            pl.BlockSpec((D, D), full),
            pl.BlockSpec((1, D), full),
            pl.BlockSpec((D, D), full),
            pl.BlockSpec((1, D), full),
        ],
        out_specs=pl.BlockSpec((1, D), full),
        out_shape=jax.ShapeDtypeStruct((1, D), jnp.float32),
        scratch_shapes=[pltpu.VMEM((1, D), jnp.float32)],
    )(p, w2, b2.reshape(1, D),
      wd1, bd1.reshape(1, D), wd2, bd2.reshape(1, D))


def kernel(node_features, edge_index, W1, b1, W2, b2, Wd1, bd1, Wd2, bd2):
    npad = EDGES_PAD - N_EDGES
    src = jnp.concatenate(
        [edge_index[0].astype(jnp.int32),
         jnp.zeros((npad,), jnp.int32)]).reshape(NS, NCHUNK, CHUNK)
    dst = jnp.concatenate(
        [edge_index[1].astype(jnp.int32),
         jnp.full((npad,), DST_PAD, jnp.int32)]).reshape(NS, NBLK, BLK, CHUNK)

    h1 = _tc_layer(_sc_spmm(node_features, src, dst), W1, b1)
    policy = _tc_head(_sc_spmm(h1, src, dst), W2, b2, Wd1, bd1, Wd2, bd2)
    return policy.reshape(D)


# speedup vs baseline: 4.4437x; 1.0124x over previous
import jax
import jax.numpy as jnp
from jax import lax
from jax.experimental import pallas as pl
from jax.experimental.pallas import tpu as pltpu
from jax.experimental.pallas import tpu_sc as plsc

N_NODES = 10000
N_EDGES = 320000
D = 128

NC = 2
NS = 16
XW = D // NC
N_PAD = 10240
ACC_ROWS = N_PAD
CHUNK = 128
NCHUNK = 160
EDGES_PAD = NS * NCHUNK * CHUNK
DST_PAD = 10016
NBUF = 2
GLEAD = 2
BLK = 8
NBLK = NCHUNK // BLK
ZCH = 64
WB_ROWS = 128


def _spmm_body(src_hbm, dst_hbm, x2_hbm, out_hbm,
               idx_s, dblk0, dblk1, rows0, rows1,
               acc_sp, x_sp, g0, g1, d0, d1):
    c = lax.axis_index("c")
    s = lax.axis_index("s")
    bufs = [rows0, rows1]
    gsems = [g0, g1]
    dblks = [dblk0, dblk1]
    dsems = [d0, d1]
    col0 = c * XW

    zero16 = jnp.zeros((16,), jnp.float32)
    for r in range(ZCH):
        for l in range(XW // 16):
            rows0[r, pl.ds(l * 16, 16)] = zero16
    zsrc = rows0.at[pl.ds(0, ZCH)]

    def zero_step(k, _):
        pltpu.sync_copy(zsrc,
                        acc_sp.at[pl.ds((s * (N_PAD // NS // ZCH) + k) * ZCH,
                                        ZCH)])
        return 0
    lax.fori_loop(0, N_PAD // NS // ZCH, zero_step, 0)

    def xst_step(k, _):
        base = s * (N_PAD // NS) + k * WB_ROWS

        @pl.when(base + WB_ROWS <= N_NODES)
        def _():
            pltpu.sync_copy(x2_hbm.at[pl.ds(base, WB_ROWS),
                                      pl.ds(col0, XW)],
                            x_sp.at[pl.ds(base, WB_ROWS)])
        return 0
    lax.fori_loop(0, N_PAD // NS // WB_ROWS, xst_step, 0)

    @pl.when(s == NS - 1)
    def _():
        xr = N_NODES % WB_ROWS
        pltpu.sync_copy(x2_hbm.at[pl.ds(N_NODES - xr, xr),
                                  pl.ds(col0, XW)],
                        x_sp.at[pl.ds(N_NODES - xr, xr)])

    pltpu.sync_copy(src_hbm.at[s], idx_s)
    pltpu.async_copy(dst_hbm.at[s, 0], dblk0, d0)

    plsc.subcore_barrier()

    for k in range(NBUF):
        pltpu.async_copy(x_sp.at[idx_s.at[k]], bufs[k], gsems[k])

    def super_step(sb, _):
        for p in range(2):
            b = 2 * sb + p
            pltpu.make_async_copy(dst_hbm.at[s, b], dblks[p],
                                  dsems[p]).wait()
            for q in range(BLK):
                j = b * BLK + q
                k = q % NBUF
                pltpu.make_async_copy(x_sp.at[idx_s.at[j]], bufs[k],
                                      gsems[k]).wait()
                pltpu.sync_copy(bufs[k], acc_sp.at[dblks[p].at[q]],
                                add=True)

                @pl.when(j + NBUF < NCHUNK)
                def _():
                    pltpu.async_copy(x_sp.at[idx_s.at[j + NBUF]],
                                     bufs[k], gsems[k])

                if q == 2:
                    @pl.when(b + 1 < NBLK)
                    def _():
                        pltpu.async_copy(dst_hbm.at[s, b + 1],
                                         dblks[1 - p], dsems[1 - p])
        return 0
    lax.fori_loop(0, NBLK // 2, super_step, 0)

    plsc.subcore_barrier()

    wsrc = rows0.at[pl.ds(0, WB_ROWS)]

    def wb_step(k, _):
        base = s * (N_PAD // NS) + k * WB_ROWS

        @pl.when(base + WB_ROWS <= N_NODES)
        def _():
            pltpu.sync_copy(acc_sp.at[pl.ds(base, WB_ROWS)], wsrc)
            pltpu.sync_copy(wsrc, out_hbm.at[pl.ds(base, WB_ROWS),
                                             pl.ds(col0, XW)])
        return 0
    lax.fori_loop(0, N_PAD // NS // WB_ROWS, wb_step, 0)

    @pl.when(s == NS - 1)
    def _():
        nr = N_NODES % WB_ROWS
        tail = rows0.at[pl.ds(0, nr)]
        pltpu.sync_copy(acc_sp.at[pl.ds(N_NODES - nr, nr)], tail)
        pltpu.sync_copy(tail, out_hbm.at[pl.ds(N_NODES - nr, nr),
                                         pl.ds(col0, XW)])


def _sc_spmm(x2, src3, dst4):
    mesh = plsc.VectorSubcoreMesh(core_axis_name="c", subcore_axis_name="s")
    rowbuf = pltpu.VMEM((CHUNK, XW), jnp.float32)
    dma = pltpu.SemaphoreType.DMA
    return pl.kernel(
        _spmm_body,
        out_type=jax.ShapeDtypeStruct((N_NODES, D), jnp.float32),
        mesh=mesh,
        compiler_params=pltpu.CompilerParams(use_tc_tiling_on_sc=False),
        scratch_types=[
            pltpu.VMEM((NCHUNK, CHUNK), jnp.int32),
            pltpu.VMEM((BLK, CHUNK), jnp.int32),
            pltpu.VMEM((BLK, CHUNK), jnp.int32),
        ] + [rowbuf] * NBUF + [
            pltpu.VMEM_SHARED((ACC_ROWS, XW), jnp.float32),
            pltpu.VMEM_SHARED((N_NODES, XW), jnp.float32),
        ] + [dma] * (NBUF + 2),
    )(src3, dst4, x2)


ROW_BLK = 1000


def _layer_body(p_ref, w_ref, b_ref, out_ref):
    y = lax.dot_general(p_ref[...], w_ref[...], (((1,), (1,)), ((), ())),
                        preferred_element_type=jnp.float32)
    out_ref[...] = jnp.maximum(y + b_ref[...], 0.0)


def _tc_layer(p, w, b):
    grid = N_NODES // ROW_BLK
    return pl.pallas_call(
        _layer_body,
        grid=(grid,),
        in_specs=[
            pl.BlockSpec((ROW_BLK, D), lambda i: (i, 0)),
            pl.BlockSpec((D, D), lambda i: (0, 0)),
            pl.BlockSpec((1, D), lambda i: (0, 0)),
        ],
        out_specs=pl.BlockSpec((ROW_BLK, D), lambda i: (i, 0)),
        out_shape=jax.ShapeDtypeStruct((N_NODES, D), jnp.float32),
    )(p, w, b.reshape(1, D))


def _head_body(p_ref, w2_ref, b2_ref, wd1_ref, bd1_ref,
               wd2_ref, bd2_ref, out_ref, acc_ref):
    i = pl.program_id(0)

    @pl.when(i == 0)
    def _():
        acc_ref[...] = jnp.zeros_like(acc_ref)

    y = lax.dot_general(p_ref[...], w2_ref[...], (((1,), (1,)), ((), ())),
                        preferred_element_type=jnp.float32)
    h2 = jnp.maximum(y + b2_ref[...], 0.0)
    acc_ref[...] += jnp.sum(h2, axis=0, keepdims=True)

    @pl.when(i == pl.num_programs(0) - 1)
    def _():
        emb = acc_ref[...] * (1.0 / N_NODES)
        d = lax.dot_general(emb, wd1_ref[...], (((1,), (1,)), ((), ())),
                            preferred_element_type=jnp.float32)
        d = jnp.maximum(d + bd1_ref[...], 0.0)
        z = lax.dot_general(d, wd2_ref[...], (((1,), (1,)), ((), ())),
                            preferred_element_type=jnp.float32)
        out_ref[...] = jax.nn.sigmoid(z + bd2_ref[...])


def _tc_head(p, w2, b2, wd1, bd1, wd2, bd2):
    grid = N_NODES // ROW_BLK
    full = lambda i: (0, 0)
    return pl.pallas_call(
        _head_body,
        grid=(grid,),
        in_specs=[
            pl.BlockSpec((ROW_BLK, D), lambda i: (i, 0)),
            pl.BlockSpec((D, D), full),
            pl.BlockSpec((1, D), full),
            pl.BlockSpec((D, D), full),
            pl.BlockSpec((1, D), full),
            pl.BlockSpec((D, D), full),
            pl.BlockSpec((1, D), full),
        ],
        out_specs=pl.BlockSpec((1, D), full),
        out_shape=jax.ShapeDtypeStruct((1, D), jnp.float32),
        scratch_shapes=[pltpu.VMEM((1, D), jnp.float32)],
    )(p, w2, b2.reshape(1, D),
      wd1, bd1.reshape(1, D), wd2, bd2.reshape(1, D))


def kernel(node_features, edge_index, W1, b1, W2, b2, Wd1, bd1, Wd2, bd2):
    npad = EDGES_PAD - N_EDGES
    src = jnp.concatenate(
        [edge_index[0].astype(jnp.int32),
         jnp.zeros((npad,), jnp.int32)]).reshape(NS, NCHUNK, CHUNK)
    dst = jnp.concatenate(
        [edge_index[1].astype(jnp.int32),
         jnp.full((npad,), DST_PAD, jnp.int32)]).reshape(NS, NBLK, BLK, CHUNK)

    h1 = _tc_layer(_sc_spmm(node_features, src, dst), W1, b1)
    policy = _tc_head(_sc_spmm(h1, src, dst), W2, b2, Wd1, bd1, Wd2, bd2)
    return policy.reshape(D)
